# baseline (speedup 1.0000x reference)
import functools
import jax
import jax.numpy as jnp
import numpy as np
from jax import lax
from jax.experimental import pallas as pl
from jax.experimental.pallas import tpu as pltpu
from jax.experimental.pallas import tpu_sc as plsc

N = 50000
E = 800000
B = 64
H1 = 4
C1 = 64
C2 = 64

EP = 819200
TBLK = 2048
BLK_N = 2000

NC = 2
NS = 16
SCC = 128
SROW = 8
NPAD = 51200
_SC_MESH = dict(core_axis_name="c", subcore_axis_name="s", num_cores=NC,
                num_subcores=NS)
_SC_PARAMS = pltpu.CompilerParams(use_tc_tiling_on_sc=False)

_F32 = jnp.float32


def _lane_sum(v):
    t = [v[j] for j in range(16)]
    while len(t) > 1:
        t = [t[a] + t[a + 1] for a in range(0, len(t), 2)]
    return t[0]


def _sc_gather_x(x_hbm, s2d_hbm, d2d_hbm, us_hbm, ud_hbm, ivs, ivd, us, ud):
    c = lax.axis_index("c")
    s = lax.axis_index("s")
    wid = c * NS + s
    e_pt = EP // (NC * NS)
    base = wid * e_pt

    def _outer(k, _):
        r0 = pl.multiple_of(base // SCC + k * SROW, 8)
        e0 = pl.multiple_of(base + k * SROW * SCC, 8)
        pltpu.sync_copy(s2d_hbm.at[pl.ds(r0, SROW)], ivs)
        pltpu.sync_copy(d2d_hbm.at[pl.ds(r0, SROW)], ivd)
        for j in range(SROW):
            pltpu.sync_copy(x_hbm.at[ivs.at[j]], us.at[pl.ds(j * SCC, SCC)])
            pltpu.sync_copy(x_hbm.at[ivd.at[j]], ud.at[pl.ds(j * SCC, SCC)])
        pltpu.sync_copy(us, us_hbm.at[pl.ds(e0, SROW * SCC)])
        pltpu.sync_copy(ud, ud_hbm.at[pl.ds(e0, SROW * SCC)])
        return 0

    lax.fori_loop(0, e_pt // (SROW * SCC), _outer, 0)


_gx = functools.partial(
    pl.kernel,
    out_type=[jax.ShapeDtypeStruct((EP, 8), _F32) for _ in range(2)],
    mesh=plsc.VectorSubcoreMesh(**_SC_MESH),
    compiler_params=_SC_PARAMS,
    scratch_types=[
        pltpu.VMEM((SROW, SCC), jnp.int32),
        pltpu.VMEM((SROW, SCC), jnp.int32),
        pltpu.VMEM((SROW * SCC, 8), _F32),
        pltpu.VMEM((SROW * SCC, 8), _F32),
    ],
)(_sc_gather_x)


def _sc_scatter16(p_hbm, dst_hbm, out_hbm, acc, iv, pv, rv):
    c = lax.axis_index("c")
    s = lax.axis_index("s")
    rows_pt = NPAD // NS
    e_pt = EP // (NC * NS)
    n_outer = e_pt // (SROW * SCC)

    def _zero_row(i, _):
        rv[i, :] = jnp.zeros((16,), _F32)
        return 0

    lax.fori_loop(0, rows_pt, _zero_row, 0)
    o0 = pl.multiple_of(s * rows_pt, 8)
    pltpu.sync_copy(rv, acc.at[pl.ds(o0, rows_pt)])
    plsc.subcore_barrier()

    base_e = (c * NS + s) * e_pt

    def _outer(k, _):
        r0 = pl.multiple_of(base_e // SCC + k * SROW, 8)
        e0 = pl.multiple_of(base_e + k * SROW * SCC, 8)
        pltpu.sync_copy(dst_hbm.at[pl.ds(r0, SROW)], iv)
        pltpu.sync_copy(p_hbm.at[pl.ds(e0, SROW * SCC)], pv)

        def _inner(j, _):
            pltpu.sync_copy(pv.at[pl.ds(j * SCC, SCC)], acc.at[iv.at[j]],
                            add=True)
            return 0

        lax.fori_loop(0, SROW, _inner, 0)
        return 0

    lax.fori_loop(0, n_outer, _outer, 0)
    plsc.subcore_barrier()
    pltpu.sync_copy(acc.at[pl.ds(o0, rows_pt)], rv)
    pltpu.sync_copy(rv, out_hbm.at[c, pl.ds(o0, rows_pt)])


_l1_scatter = functools.partial(
    pl.kernel,
    out_type=jax.ShapeDtypeStruct((NC, NPAD, 16), _F32),
    mesh=plsc.VectorSubcoreMesh(**_SC_MESH),
    compiler_params=_SC_PARAMS,
    scratch_types=[
        pltpu.VMEM_SHARED((NPAD, 16), _F32),
        pltpu.VMEM((SROW, SCC), jnp.int32),
        pltpu.VMEM((SROW * SCC, 16), _F32),
        pltpu.VMEM((NPAD // NS, 16), _F32),
    ],
)(_sc_scatter16)


def _sc_edge2(xl2a_hbm, xl2b_hbm, xr2a_hbm, xr2b_hbm, att_hbm, s2d_hbm,
              d2d_hbm, lo_hbm, attv, ivs, ivd, av0, av1, bv0, bv1, lv):
    c = lax.axis_index("c")
    s = lax.axis_index("s")
    wid = c * NS + s
    e_pt = EP // (NC * NS)
    base = wid * e_pt
    pltpu.sync_copy(att_hbm, attv)

    def _outer(k, _):
        r0 = pl.multiple_of(base // SCC + k * SROW, 8)
        pltpu.sync_copy(s2d_hbm.at[pl.ds(r0, SROW)], ivs)
        pltpu.sync_copy(d2d_hbm.at[pl.ds(r0, SROW)], ivd)

        def _half(h, _):
            for j in range(4):
                row = h * 4 + j
                pltpu.sync_copy(xl2a_hbm.at[ivs.at[row]],
                                av0.at[pl.ds(j * SCC, SCC)])
                pltpu.sync_copy(xl2b_hbm.at[ivs.at[row]],
                                av1.at[pl.ds(j * SCC, SCC)])
                pltpu.sync_copy(xr2a_hbm.at[ivd.at[row]],
                                bv0.at[pl.ds(j * SCC, SCC)])
                pltpu.sync_copy(xr2b_hbm.at[ivd.at[row]],
                                bv1.at[pl.ds(j * SCC, SCC)])

            def _grp(g, _):
                a0 = attv[pl.ds(0, 16)]
                a1 = attv[pl.ds(16, 16)]
                a2 = attv[pl.ds(32, 16)]
                a3 = attv[pl.ds(48, 16)]
                ioa = lax.iota(jnp.int32, 16)
                accv = jnp.zeros((16,), _F32)
                for i in range(16):
                    e = g * 16 + i
                    z0 = av0[e, pl.ds(0, 16)] + bv0[e, pl.ds(0, 16)]
                    z1 = av0[e, pl.ds(16, 16)] + bv0[e, pl.ds(16, 16)]
                    z2 = av1[e, pl.ds(0, 16)] + bv1[e, pl.ds(0, 16)]
                    z3 = av1[e, pl.ds(16, 16)] + bv1[e, pl.ds(16, 16)]
                    m = (jnp.where(z0 > 0, z0, 0.2 * z0) * a0
                         + jnp.where(z1 > 0, z1, 0.2 * z1) * a1
                         + jnp.where(z2 > 0, z2, 0.2 * z2) * a2
                         + jnp.where(z3 > 0, z3, 0.2 * z3) * a3)
                    logit = _lane_sum(m)
                    accv = jnp.where(ioa == i, jnp.full((16,), logit, _F32),
                                     accv)
                gidv = base + k * SROW * SCC + h * 512 + g * 16 + ioa
                lv[pl.ds(g * 16, 16)] = jnp.where(gidv < E, accv, -1e30)
                return 0

            lax.fori_loop(0, 32, _grp, 0)
            e0 = pl.multiple_of(base + k * SROW * SCC + h * 512, 8)
            pltpu.sync_copy(lv, lo_hbm.at[pl.ds(e0, 512)])
            return 0

        lax.fori_loop(0, 2, _half, 0)
        return 0

    lax.fori_loop(0, e_pt // (SROW * SCC), _outer, 0)


_l2_logits = functools.partial(
    pl.kernel,
    out_type=jax.ShapeDtypeStruct((EP,), _F32),
    mesh=plsc.VectorSubcoreMesh(**_SC_MESH),
    compiler_params=_SC_PARAMS,
    scratch_types=[
        pltpu.VMEM((C2,), _F32),
        pltpu.VMEM((SROW, SCC), jnp.int32),
        pltpu.VMEM((SROW, SCC), jnp.int32),
        pltpu.VMEM((512, 32), _F32),
        pltpu.VMEM((512, 32), _F32),
        pltpu.VMEM((512, 32), _F32),
        pltpu.VMEM((512, 32), _F32),
        pltpu.VMEM((512,), _F32),
    ],
)(_sc_edge2)


ZCH = 160


def _sc_scatter2(xl2a_hbm, xl2b_hbm, lo_hbm, d2d_hbm, o32_hbm,
                 acc32, av, rv, lv, iv, zbuf):
    c = lax.axis_index("c")
    s = lax.axis_index("s")
    rows_pt = NPAD // NS

    def _zrow(i, _):
        zbuf[i, pl.ds(0, 16)] = jnp.zeros((16,), _F32)
        zbuf[i, pl.ds(16, 16)] = jnp.zeros((16,), _F32)
        return 0

    lax.fori_loop(0, ZCH, _zrow, 0)
    o0 = pl.multiple_of(s * rows_pt, 8)
    for m in range(rows_pt // ZCH):
        pltpu.sync_copy(zbuf, acc32.at[pl.ds(o0 + m * ZCH, ZCH)])

    plsc.subcore_barrier()

    e_pt = EP // NS
    base = s * e_pt

    def _outer(k, _):
        r0 = pl.multiple_of(base // SCC + k * SROW, 8)
        e0 = pl.multiple_of(base + k * SROW * SCC, 8)
        pltpu.sync_copy(d2d_hbm.at[pl.ds(r0, SROW)], iv)
        pltpu.sync_copy(lo_hbm.at[pl.ds(e0, SROW * SCC)], lv)

        def _quarter(q, _):
            j0 = q * 2
            for jj in range(2):
                @pl.when(c == 0)
                def _():
                    pltpu.sync_copy(xl2a_hbm.at[iv.at[j0 + jj]],
                                    av.at[pl.ds(jj * SCC, SCC)])

                @pl.when(c == 1)
                def _():
                    pltpu.sync_copy(xl2b_hbm.at[iv.at[j0 + jj]],
                                    av.at[pl.ds(jj * SCC, SCC)])

            def _grp(g, _):
                wv = jnp.exp(lv[pl.ds(q * 2 * SCC + g * 16, 16)])
                for i in range(16):
                    e = g * 16 + i
                    wb = jnp.full((16,), wv[i], _F32)
                    rv[e, pl.ds(0, 16)] = av[e, pl.ds(0, 16)] * wb
                    rv[e, pl.ds(16, 16)] = av[e, pl.ds(16, 16)] * wb
                return 0

            lax.fori_loop(0, 2 * SCC // 16, _grp, 0)
            for jj in range(2):
                pltpu.sync_copy(rv.at[pl.ds(jj * SCC, SCC)],
                                acc32.at[iv.at[j0 + jj]], add=True)
            return 0

        lax.fori_loop(0, SROW // 2, _quarter, 0)
        return 0

    lax.fori_loop(0, e_pt // (SROW * SCC), _outer, 0)
    plsc.subcore_barrier()
    for m in range(rows_pt // ZCH):
        pltpu.sync_copy(acc32.at[pl.ds(o0 + m * ZCH, ZCH)], zbuf)
        pltpu.sync_copy(zbuf, o32_hbm.at[c, pl.ds(o0 + m * ZCH, ZCH)])


_l2_scatter = functools.partial(
    pl.kernel,
    out_type=jax.ShapeDtypeStruct((NC, NPAD, 32), _F32),
    mesh=plsc.VectorSubcoreMesh(**_SC_MESH),
    compiler_params=_SC_PARAMS,
    scratch_types=[
        pltpu.VMEM_SHARED((NPAD, 32), _F32),
        pltpu.VMEM((2 * SCC, 32), _F32),
        pltpu.VMEM((2 * SCC, 32), _F32),
        pltpu.VMEM((SROW * SCC,), _F32),
        pltpu.VMEM((SROW, SCC), jnp.int32),
        pltpu.VMEM((ZCH, 32), _F32),
    ],
)(_sc_scatter2)


def _sc_scatterd(lo_hbm, d2d_hbm, od_hbm, accd, iv, lv, dv, rb):
    c = lax.axis_index("c")
    s = lax.axis_index("s")
    rows_pt = NPAD // NS

    def _zrow(i, _):
        rb[pl.ds(i * 16, 16)] = jnp.zeros((16,), _F32)
        return 0

    lax.fori_loop(0, rows_pt // 16, _zrow, 0)
    o0 = pl.multiple_of(s * rows_pt, 8)
    pltpu.sync_copy(rb, accd.at[pl.ds(o0, rows_pt)])
    plsc.subcore_barrier()

    e_pt = EP // (NC * NS)
    base = (c * NS + s) * e_pt

    def _outer(k, _):
        r0 = pl.multiple_of(base // SCC + k * SROW, 8)
        e0 = pl.multiple_of(base + k * SROW * SCC, 8)
        pltpu.sync_copy(d2d_hbm.at[pl.ds(r0, SROW)], iv)
        pltpu.sync_copy(lo_hbm.at[pl.ds(e0, SROW * SCC)], lv)

        def _grp(g, _):
            dv[pl.ds(g * 16, 16)] = jnp.exp(lv[pl.ds(g * 16, 16)])
            return 0

        lax.fori_loop(0, SROW * SCC // 16, _grp, 0)
        for j in range(SROW):
            pltpu.sync_copy(dv.at[pl.ds(j * SCC, SCC)], accd.at[iv.at[j]],
                            add=True)
        return 0

    lax.fori_loop(0, e_pt // (SROW * SCC), _outer, 0)
    plsc.subcore_barrier()
    pltpu.sync_copy(accd.at[pl.ds(o0, rows_pt)], rb)
    pltpu.sync_copy(rb, od_hbm.at[c, pl.ds(o0, rows_pt)])


_l2_scatterd = functools.partial(
    pl.kernel,
    out_type=jax.ShapeDtypeStruct((NC, NPAD), _F32),
    mesh=plsc.VectorSubcoreMesh(**_SC_MESH),
    compiler_params=_SC_PARAMS,
    scratch_types=[
        pltpu.VMEM_SHARED((NPAD,), _F32),
        pltpu.VMEM((SROW, SCC), jnp.int32),
        pltpu.VMEM((SROW * SCC,), _F32),
        pltpu.VMEM((SROW * SCC,), _F32),
        pltpu.VMEM((NPAD // NS,), _F32),
    ],
)(_sc_scatterd)


def _k1_edge1(us_ref, ud_ref, w4_ref, b4_ref, asel_ref, rw_ref, ru_ref, p_ref):
    i = pl.program_id(0)
    u = jnp.concatenate([us_ref[:, 0:2], ud_ref[:, 0:2]], axis=1)
    z = jnp.dot(u, w4_ref[...], preferred_element_type=_F32) + b4_ref[...]
    e = jnp.where(z > 0, z, 0.2 * z)
    logits = jnp.dot(e, asel_ref[...], preferred_element_type=_F32)
    rid = i * TBLK + lax.broadcasted_iota(jnp.int32, (TBLK, 1), 0)
    w = jnp.exp(logits) * (rid < E).astype(_F32)
    w_rep = jnp.dot(w, rw_ref[...], preferred_element_type=_F32)
    u_til = jnp.dot(u, ru_ref[...], preferred_element_type=_F32)
    p_ref[:, 0:4] = w
    p_ref[:, 4:12] = w_rep * u_til
    p_ref[:, 12:16] = jnp.zeros_like(p_ref[:, 12:16])


def _k2_node1(acc_ref, m16_ref, k4_ref, bias1_ref, wl2a_ref, wl2b_ref,
              wr2a_ref, wr2b_ref, bl2_ref, br2_ref,
              xl2a_ref, xl2b_ref, xr2a_ref, xr2b_ref):
    acc3 = acc_ref[...]
    acc = acc3[0] + acc3[1]
    denom = acc[:, 0:4]
    out1_pre = jnp.dot(acc, m16_ref[...], preferred_element_type=_F32)
    recip = 1.0 / (denom + 1e-16)
    bcast = jnp.dot(recip, k4_ref[...], preferred_element_type=_F32)
    h1 = jnp.maximum(out1_pre * bcast + bias1_ref[...], 0.0)
    xl2a_ref[...] = jnp.dot(h1, wl2a_ref[...], preferred_element_type=_F32) + bl2_ref[:, 0:32]
    xl2b_ref[...] = jnp.dot(h1, wl2b_ref[...], preferred_element_type=_F32) + bl2_ref[:, 32:64]
    xr2a_ref[...] = jnp.dot(h1, wr2a_ref[...], preferred_element_type=_F32) + br2_ref[:, 0:32]
    xr2b_ref[...] = jnp.dot(h1, wr2b_ref[...], preferred_element_type=_F32) + br2_ref[:, 32:64]


def _k4_finish(o32_ref, od_ref, batch_ref, bias2_ref, bk_ref, fw1_ref, fb1_ref,
               fw2_ref, fb2_ref, fw3_ref, fb3_ref, out_ref, sums_ref, cnt_ref):
    i = pl.program_id(0)
    nblk = pl.num_programs(0)

    @pl.when(i == 0)
    def _init():
        sums_ref[...] = jnp.zeros_like(sums_ref)
        cnt_ref[...] = jnp.zeros_like(cnt_ref)

    o32 = o32_ref[...]
    od3 = od_ref[...]
    recip = 1.0 / (od3[0] + od3[1] + 1e-16)
    h2a = o32[0] * recip + bias2_ref[:, 0:32]
    h2b = o32[1] * recip + bias2_ref[:, 32:64]
    bvec = batch_ref[0, :, :]
    gids = lax.broadcasted_iota(jnp.int32, (B, BLK_N), 0)
    oh = (gids == bvec).astype(_F32)
    sums_ref[:, 0:32] += jnp.dot(oh, h2a, preferred_element_type=_F32)
    sums_ref[:, 32:64] += jnp.dot(oh, h2b, preferred_element_type=_F32)
    cnt_ref[:, 0:1] += jnp.sum(oh, axis=1, keepdims=True)

    @pl.when(i == nblk - 1)
    def _fin():
        ge = sums_ref[...] / jnp.maximum(cnt_ref[:, 0:1], 1.0)
        bk = bk_ref[...]
        c = ge @ fw1_ref[0:64, :] + bk @ fw1_ref[64:65, :] + fb1_ref[...]
        c = jnp.maximum(c, 0.0)
        c = jnp.maximum(c @ fw2_ref[...] + fb2_ref[...], 0.0)
        out_ref[...] = bk + c @ fw3_ref[...] + fb3_ref[...]


def _full(shape):
    return pl.BlockSpec(shape, lambda i: tuple(0 for _ in shape))


def kernel(x, edge_index, batch, baseline_k, Wl1, bl1, Wr1, br1, att1, bias1,
           Wl2, bl2, Wr2, br2, att2, bias2, fw1, fb1, fw2, fb2, fw3, fb3):
    src = edge_index[0]
    dst = edge_index[1]

    kmask = np.zeros((H1, H1 * C1), np.float32)
    for h in range(H1):
        kmask[h, h * C1:(h + 1) * C1] = 1.0
    kmask_j = jnp.asarray(kmask)
    W4 = jnp.concatenate([Wl1, Wr1], axis=0)
    b4 = (bl1 + br1).reshape(1, H1 * C1)
    Asel = att1.reshape(H1 * C1, 1) * kmask_j.T
    rows = [bl1 * kmask[h] for h in range(H1)]
    for h in range(H1):
        for k in range(2):
            rows.append(Wl1[k] * kmask[h])
    for _ in range(4):
        rows.append(jnp.zeros((H1 * C1,), _F32))
    M16 = jnp.stack(rows, axis=0)
    rw = np.zeros((4, 8), np.float32)
    ru = np.zeros((4, 8), np.float32)
    for h in range(H1):
        for k in range(2):
            rw[h, 2 * h + k] = 1.0
            ru[k, 2 * h + k] = 1.0
    rw_j, ru_j = jnp.asarray(rw), jnp.asarray(ru)

    srcp = jnp.pad(src, (0, EP - E))
    dstp = jnp.pad(dst, (0, EP - E))
    src2d = srcp.reshape(EP // SCC, SCC)
    dst2d = dstp.reshape(EP // SCC, SCC)

    x8 = jnp.pad(x, ((0, 0), (0, 6)))
    usrc, udst = _gx(x8, src2d, dst2d)

    p = pl.pallas_call(
        _k1_edge1,
        grid=(EP // TBLK,),
        in_specs=[pl.BlockSpec((TBLK, 8), lambda i: (i, 0))] * 2 + [
            _full((4, H1 * C1)), _full((1, H1 * C1)),
            _full((H1 * C1, 4)), _full((4, 8)), _full((4, 8))],
        out_specs=pl.BlockSpec((TBLK, 16), lambda i: (i, 0)),
        out_shape=jax.ShapeDtypeStruct((EP, 16), _F32),
    )(usrc, udst, W4, b4, Asel, rw_j, ru_j)

    acc1 = _l1_scatter(p, dst2d)

    xl2a, xl2b, xr2a, xr2b = pl.pallas_call(
        _k2_node1,
        grid=(N // BLK_N,),
        in_specs=[pl.BlockSpec((NC, BLK_N, 16), lambda i: (0, i, 0)),
                  _full((16, H1 * C1)), _full((4, H1 * C1)), _full((1, H1 * C1)),
                  _full((H1 * C1, 32)), _full((H1 * C1, 32)),
                  _full((H1 * C1, 32)), _full((H1 * C1, 32)),
                  _full((1, C2)), _full((1, C2))],
        out_specs=[pl.BlockSpec((BLK_N, 32), lambda i: (i, 0))] * 4,
        out_shape=[jax.ShapeDtypeStruct((N, 32), _F32)] * 4,
    )(acc1, M16, kmask_j, bias1.reshape(1, -1),
      Wl2[:, 0:32], Wl2[:, 32:64], Wr2[:, 0:32], Wr2[:, 32:64],
      bl2.reshape(1, -1), br2.reshape(1, -1))

    logits = _l2_logits(xl2a, xl2b, xr2a, xr2b, att2.reshape(-1),
                        src2d, dst2d)

    o32 = _l2_scatter(xl2a, xl2b, logits, dst2d)
    od = _l2_scatterd(logits, dst2d)

    batch3d = batch.reshape(N // BLK_N, 1, BLK_N)
    od3 = od.reshape(NC, NPAD, 1)
    out = pl.pallas_call(
        _k4_finish,
        grid=(N // BLK_N,),
        in_specs=[pl.BlockSpec((NC, BLK_N, 32), lambda i: (0, i, 0)),
                  pl.BlockSpec((NC, BLK_N, 1), lambda i: (0, i, 0)),
                  pl.BlockSpec((1, 1, BLK_N), lambda i: (i, 0, 0)),
                  _full((1, C2)), _full((B, 1)),
                  _full((C2 + 1, 32)), _full((1, 32)),
                  _full((32, 16)), _full((1, 16)),
                  _full((16, 1)), _full((1, 1))],
        out_specs=pl.BlockSpec((B, 1), lambda i: (0, 0)),
        out_shape=jax.ShapeDtypeStruct((B, 1), _F32),
        scratch_shapes=[pltpu.VMEM((B, C2), _F32), pltpu.VMEM((B, 128), _F32)],
    )(o32, od3, batch3d, bias2.reshape(1, -1), baseline_k,
      fw1, fb1.reshape(1, -1), fw2, fb2.reshape(1, -1), fw3, fb3.reshape(1, -1))
    return out


# speedup vs baseline: 1.2403x; 1.2403x over previous
import functools
import jax
import jax.numpy as jnp
import numpy as np
from jax import lax
from jax.experimental import pallas as pl
from jax.experimental.pallas import tpu as pltpu
from jax.experimental.pallas import tpu_sc as plsc

N = 50000
E = 800000
B = 64
H1 = 4
C1 = 64
C2 = 64

EP = 819200
TBLK = 2048
BLK_N = 2000

NC = 2
NS = 16
SCC = 128
SROW = 8
NPAD = 51200
_SC_MESH = dict(core_axis_name="c", subcore_axis_name="s", num_cores=NC,
                num_subcores=NS)
_SC_PARAMS = pltpu.CompilerParams(use_tc_tiling_on_sc=False)

_F32 = jnp.float32


def _lane_sum(v):
    t = [v[j] for j in range(16)]
    while len(t) > 1:
        t = [t[a] + t[a + 1] for a in range(0, len(t), 2)]
    return t[0]


def _sc_gather_x(x_hbm, s2d_hbm, d2d_hbm, us_hbm, ud_hbm, ivs, ivd, us, ud):
    c = lax.axis_index("c")
    s = lax.axis_index("s")
    wid = c * NS + s
    e_pt = EP // (NC * NS)
    base = wid * e_pt

    def _outer(k, _):
        r0 = pl.multiple_of(base // SCC + k * SROW, 8)
        e0 = pl.multiple_of(base + k * SROW * SCC, 8)
        pltpu.sync_copy(s2d_hbm.at[pl.ds(r0, SROW)], ivs)
        pltpu.sync_copy(d2d_hbm.at[pl.ds(r0, SROW)], ivd)
        for j in range(SROW):
            pltpu.sync_copy(x_hbm.at[ivs.at[j]], us.at[pl.ds(j * SCC, SCC)])
            pltpu.sync_copy(x_hbm.at[ivd.at[j]], ud.at[pl.ds(j * SCC, SCC)])
        pltpu.sync_copy(us, us_hbm.at[pl.ds(e0, SROW * SCC)])
        pltpu.sync_copy(ud, ud_hbm.at[pl.ds(e0, SROW * SCC)])
        return 0

    lax.fori_loop(0, e_pt // (SROW * SCC), _outer, 0)


_gx = functools.partial(
    pl.kernel,
    out_type=[jax.ShapeDtypeStruct((EP, 8), _F32) for _ in range(2)],
    mesh=plsc.VectorSubcoreMesh(**_SC_MESH),
    compiler_params=_SC_PARAMS,
    scratch_types=[
        pltpu.VMEM((SROW, SCC), jnp.int32),
        pltpu.VMEM((SROW, SCC), jnp.int32),
        pltpu.VMEM((SROW * SCC, 8), _F32),
        pltpu.VMEM((SROW * SCC, 8), _F32),
    ],
)(_sc_gather_x)


def _sc_scatter16(p_hbm, dst_hbm, out_hbm, acc, iv, pv, rv):
    c = lax.axis_index("c")
    s = lax.axis_index("s")
    rows_pt = NPAD // NS
    e_pt = EP // (NC * NS)
    n_outer = e_pt // (SROW * SCC)

    def _zero_row(i, _):
        rv[i, :] = jnp.zeros((16,), _F32)
        return 0

    lax.fori_loop(0, rows_pt, _zero_row, 0)
    o0 = pl.multiple_of(s * rows_pt, 8)
    pltpu.sync_copy(rv, acc.at[pl.ds(o0, rows_pt)])
    plsc.subcore_barrier()

    base_e = (c * NS + s) * e_pt

    def _outer(k, _):
        r0 = pl.multiple_of(base_e // SCC + k * SROW, 8)
        e0 = pl.multiple_of(base_e + k * SROW * SCC, 8)
        pltpu.sync_copy(dst_hbm.at[pl.ds(r0, SROW)], iv)
        pltpu.sync_copy(p_hbm.at[pl.ds(e0, SROW * SCC)], pv)

        def _inner(j, _):
            pltpu.sync_copy(pv.at[pl.ds(j * SCC, SCC)], acc.at[iv.at[j]],
                            add=True)
            return 0

        lax.fori_loop(0, SROW, _inner, 0)
        return 0

    lax.fori_loop(0, n_outer, _outer, 0)
    plsc.subcore_barrier()
    pltpu.sync_copy(acc.at[pl.ds(o0, rows_pt)], rv)
    pltpu.sync_copy(rv, out_hbm.at[c, pl.ds(o0, rows_pt)])


_l1_scatter = functools.partial(
    pl.kernel,
    out_type=jax.ShapeDtypeStruct((NC, NPAD, 16), _F32),
    mesh=plsc.VectorSubcoreMesh(**_SC_MESH),
    compiler_params=_SC_PARAMS,
    scratch_types=[
        pltpu.VMEM_SHARED((NPAD, 16), _F32),
        pltpu.VMEM((SROW, SCC), jnp.int32),
        pltpu.VMEM((SROW * SCC, 16), _F32),
        pltpu.VMEM((NPAD // NS, 16), _F32),
    ],
)(_sc_scatter16)


def _sc_edge2(xl2a_hbm, xl2b_hbm, xr2a_hbm, xr2b_hbm, att_hbm, s2d_hbm,
              d2d_hbm, lo_hbm, attv, ivs, ivd, av0, av1, bv0, bv1, lv, sem):
    c = lax.axis_index("c")
    s = lax.axis_index("s")
    wid = c * NS + s
    e_pt = EP // (NC * NS)
    base = wid * e_pt
    pltpu.sync_copy(att_hbm, attv)

    def _outer(k, _):
        r0 = pl.multiple_of(base // SCC + k * SROW, 8)
        pltpu.sync_copy(s2d_hbm.at[pl.ds(r0, SROW)], ivs)
        pltpu.sync_copy(d2d_hbm.at[pl.ds(r0, SROW)], ivd)

        def _half(h, _):
            copies = []
            for j in range(4):
                row = h * 4 + j
                copies.append(pltpu.async_copy(
                    xl2a_hbm.at[ivs.at[row]], av0.at[pl.ds(j * SCC, SCC)], sem))
                copies.append(pltpu.async_copy(
                    xl2b_hbm.at[ivs.at[row]], av1.at[pl.ds(j * SCC, SCC)], sem))
                copies.append(pltpu.async_copy(
                    xr2a_hbm.at[ivd.at[row]], bv0.at[pl.ds(j * SCC, SCC)], sem))
                copies.append(pltpu.async_copy(
                    xr2b_hbm.at[ivd.at[row]], bv1.at[pl.ds(j * SCC, SCC)], sem))
            for cp in copies:
                cp.wait()

            def _grp(g, _):
                a0 = attv[pl.ds(0, 16)]
                a1 = attv[pl.ds(16, 16)]
                a2 = attv[pl.ds(32, 16)]
                a3 = attv[pl.ds(48, 16)]
                ioa = lax.iota(jnp.int32, 16)
                accv = jnp.zeros((16,), _F32)
                for i in range(16):
                    e = g * 16 + i
                    z0 = av0[e, pl.ds(0, 16)] + bv0[e, pl.ds(0, 16)]
                    z1 = av0[e, pl.ds(16, 16)] + bv0[e, pl.ds(16, 16)]
                    z2 = av1[e, pl.ds(0, 16)] + bv1[e, pl.ds(0, 16)]
                    z3 = av1[e, pl.ds(16, 16)] + bv1[e, pl.ds(16, 16)]
                    m = (jnp.where(z0 > 0, z0, 0.2 * z0) * a0
                         + jnp.where(z1 > 0, z1, 0.2 * z1) * a1
                         + jnp.where(z2 > 0, z2, 0.2 * z2) * a2
                         + jnp.where(z3 > 0, z3, 0.2 * z3) * a3)
                    logit = _lane_sum(m)
                    accv = jnp.where(ioa == i, jnp.full((16,), logit, _F32),
                                     accv)
                gidv = base + k * SROW * SCC + h * 512 + g * 16 + ioa
                lv[pl.ds(g * 16, 16)] = jnp.where(gidv < E, accv, -1e30)
                return 0

            lax.fori_loop(0, 32, _grp, 0)
            e0 = pl.multiple_of(base + k * SROW * SCC + h * 512, 8)
            pltpu.sync_copy(lv, lo_hbm.at[pl.ds(e0, 512)])
            return 0

        lax.fori_loop(0, 2, _half, 0)
        return 0

    lax.fori_loop(0, e_pt // (SROW * SCC), _outer, 0)


_l2_logits = functools.partial(
    pl.kernel,
    out_type=jax.ShapeDtypeStruct((EP,), _F32),
    mesh=plsc.VectorSubcoreMesh(**_SC_MESH),
    compiler_params=_SC_PARAMS,
    scratch_types=[
        pltpu.VMEM((C2,), _F32),
        pltpu.VMEM((SROW, SCC), jnp.int32),
        pltpu.VMEM((SROW, SCC), jnp.int32),
        pltpu.VMEM((512, 32), _F32),
        pltpu.VMEM((512, 32), _F32),
        pltpu.VMEM((512, 32), _F32),
        pltpu.VMEM((512, 32), _F32),
        pltpu.VMEM((512,), _F32),
        pltpu.SemaphoreType.DMA,
    ],
)(_sc_edge2)


ZCH = 160


def _sc_scatter2(xl2a_hbm, xl2b_hbm, lo_hbm, d2d_hbm, o32_hbm,
                 acc32, av, rv, lv, iv, zbuf):
    c = lax.axis_index("c")
    s = lax.axis_index("s")
    rows_pt = NPAD // NS

    def _zrow(i, _):
        zbuf[i, pl.ds(0, 16)] = jnp.zeros((16,), _F32)
        zbuf[i, pl.ds(16, 16)] = jnp.zeros((16,), _F32)
        return 0

    lax.fori_loop(0, ZCH, _zrow, 0)
    o0 = pl.multiple_of(s * rows_pt, 8)
    for m in range(rows_pt // ZCH):
        pltpu.sync_copy(zbuf, acc32.at[pl.ds(o0 + m * ZCH, ZCH)])

    plsc.subcore_barrier()

    e_pt = EP // NS
    base = s * e_pt

    def _outer(k, _):
        r0 = pl.multiple_of(base // SCC + k * SROW, 8)
        e0 = pl.multiple_of(base + k * SROW * SCC, 8)
        pltpu.sync_copy(d2d_hbm.at[pl.ds(r0, SROW)], iv)
        pltpu.sync_copy(lo_hbm.at[pl.ds(e0, SROW * SCC)], lv)

        def _quarter(q, _):
            j0 = q * 2
            for jj in range(2):
                @pl.when(c == 0)
                def _():
                    pltpu.sync_copy(xl2a_hbm.at[iv.at[j0 + jj]],
                                    av.at[pl.ds(jj * SCC, SCC)])

                @pl.when(c == 1)
                def _():
                    pltpu.sync_copy(xl2b_hbm.at[iv.at[j0 + jj]],
                                    av.at[pl.ds(jj * SCC, SCC)])

            def _grp(g, _):
                wv = jnp.exp(lv[pl.ds(q * 2 * SCC + g * 16, 16)])
                for i in range(16):
                    e = g * 16 + i
                    wb = jnp.full((16,), wv[i], _F32)
                    rv[e, pl.ds(0, 16)] = av[e, pl.ds(0, 16)] * wb
                    rv[e, pl.ds(16, 16)] = av[e, pl.ds(16, 16)] * wb
                return 0

            lax.fori_loop(0, 2 * SCC // 16, _grp, 0)
            for jj in range(2):
                pltpu.sync_copy(rv.at[pl.ds(jj * SCC, SCC)],
                                acc32.at[iv.at[j0 + jj]], add=True)
            return 0

        lax.fori_loop(0, SROW // 2, _quarter, 0)
        return 0

    lax.fori_loop(0, e_pt // (SROW * SCC), _outer, 0)
    plsc.subcore_barrier()
    for m in range(rows_pt // ZCH):
        pltpu.sync_copy(acc32.at[pl.ds(o0 + m * ZCH, ZCH)], zbuf)
        pltpu.sync_copy(zbuf, o32_hbm.at[c, pl.ds(o0 + m * ZCH, ZCH)])


_l2_scatter = functools.partial(
    pl.kernel,
    out_type=jax.ShapeDtypeStruct((NC, NPAD, 32), _F32),
    mesh=plsc.VectorSubcoreMesh(**_SC_MESH),
    compiler_params=_SC_PARAMS,
    scratch_types=[
        pltpu.VMEM_SHARED((NPAD, 32), _F32),
        pltpu.VMEM((2 * SCC, 32), _F32),
        pltpu.VMEM((2 * SCC, 32), _F32),
        pltpu.VMEM((SROW * SCC,), _F32),
        pltpu.VMEM((SROW, SCC), jnp.int32),
        pltpu.VMEM((ZCH, 32), _F32),
    ],
)(_sc_scatter2)


def _sc_scatterd(lo_hbm, d2d_hbm, od_hbm, accd, iv, lv, dv, rb):
    c = lax.axis_index("c")
    s = lax.axis_index("s")
    rows_pt = NPAD // NS

    def _zrow(i, _):
        rb[pl.ds(i * 16, 16)] = jnp.zeros((16,), _F32)
        return 0

    lax.fori_loop(0, rows_pt // 16, _zrow, 0)
    o0 = pl.multiple_of(s * rows_pt, 8)
    pltpu.sync_copy(rb, accd.at[pl.ds(o0, rows_pt)])
    plsc.subcore_barrier()

    e_pt = EP // (NC * NS)
    base = (c * NS + s) * e_pt

    def _outer(k, _):
        r0 = pl.multiple_of(base // SCC + k * SROW, 8)
        e0 = pl.multiple_of(base + k * SROW * SCC, 8)
        pltpu.sync_copy(d2d_hbm.at[pl.ds(r0, SROW)], iv)
        pltpu.sync_copy(lo_hbm.at[pl.ds(e0, SROW * SCC)], lv)

        def _grp(g, _):
            dv[pl.ds(g * 16, 16)] = jnp.exp(lv[pl.ds(g * 16, 16)])
            return 0

        lax.fori_loop(0, SROW * SCC // 16, _grp, 0)
        for j in range(SROW):
            pltpu.sync_copy(dv.at[pl.ds(j * SCC, SCC)], accd.at[iv.at[j]],
                            add=True)
        return 0

    lax.fori_loop(0, e_pt // (SROW * SCC), _outer, 0)
    plsc.subcore_barrier()
    pltpu.sync_copy(accd.at[pl.ds(o0, rows_pt)], rb)
    pltpu.sync_copy(rb, od_hbm.at[c, pl.ds(o0, rows_pt)])


_l2_scatterd = functools.partial(
    pl.kernel,
    out_type=jax.ShapeDtypeStruct((NC, NPAD), _F32),
    mesh=plsc.VectorSubcoreMesh(**_SC_MESH),
    compiler_params=_SC_PARAMS,
    scratch_types=[
        pltpu.VMEM_SHARED((NPAD,), _F32),
        pltpu.VMEM((SROW, SCC), jnp.int32),
        pltpu.VMEM((SROW * SCC,), _F32),
        pltpu.VMEM((SROW * SCC,), _F32),
        pltpu.VMEM((NPAD // NS,), _F32),
    ],
)(_sc_scatterd)


def _k1_edge1(us_ref, ud_ref, w4_ref, b4_ref, asel_ref, rw_ref, ru_ref, p_ref):
    i = pl.program_id(0)
    u = jnp.concatenate([us_ref[:, 0:2], ud_ref[:, 0:2]], axis=1)
    z = jnp.dot(u, w4_ref[...], preferred_element_type=_F32) + b4_ref[...]
    e = jnp.where(z > 0, z, 0.2 * z)
    logits = jnp.dot(e, asel_ref[...], preferred_element_type=_F32)
    rid = i * TBLK + lax.broadcasted_iota(jnp.int32, (TBLK, 1), 0)
    w = jnp.exp(logits) * (rid < E).astype(_F32)
    w_rep = jnp.dot(w, rw_ref[...], preferred_element_type=_F32)
    u_til = jnp.dot(u, ru_ref[...], preferred_element_type=_F32)
    p_ref[:, 0:4] = w
    p_ref[:, 4:12] = w_rep * u_til
    p_ref[:, 12:16] = jnp.zeros_like(p_ref[:, 12:16])


def _k2_node1(acc_ref, m16_ref, k4_ref, bias1_ref, wl2a_ref, wl2b_ref,
              wr2a_ref, wr2b_ref, bl2_ref, br2_ref,
              xl2a_ref, xl2b_ref, xr2a_ref, xr2b_ref):
    acc3 = acc_ref[...]
    acc = acc3[0] + acc3[1]
    denom = acc[:, 0:4]
    out1_pre = jnp.dot(acc, m16_ref[...], preferred_element_type=_F32)
    recip = 1.0 / (denom + 1e-16)
    bcast = jnp.dot(recip, k4_ref[...], preferred_element_type=_F32)
    h1 = jnp.maximum(out1_pre * bcast + bias1_ref[...], 0.0)
    xl2a_ref[...] = jnp.dot(h1, wl2a_ref[...], preferred_element_type=_F32) + bl2_ref[:, 0:32]
    xl2b_ref[...] = jnp.dot(h1, wl2b_ref[...], preferred_element_type=_F32) + bl2_ref[:, 32:64]
    xr2a_ref[...] = jnp.dot(h1, wr2a_ref[...], preferred_element_type=_F32) + br2_ref[:, 0:32]
    xr2b_ref[...] = jnp.dot(h1, wr2b_ref[...], preferred_element_type=_F32) + br2_ref[:, 32:64]


def _k4_finish(o32_ref, od_ref, batch_ref, bias2_ref, bk_ref, fw1_ref, fb1_ref,
               fw2_ref, fb2_ref, fw3_ref, fb3_ref, out_ref, sums_ref, cnt_ref):
    i = pl.program_id(0)
    nblk = pl.num_programs(0)

    @pl.when(i == 0)
    def _init():
        sums_ref[...] = jnp.zeros_like(sums_ref)
        cnt_ref[...] = jnp.zeros_like(cnt_ref)

    o32 = o32_ref[...]
    od3 = od_ref[...]
    recip = 1.0 / (od3[0] + od3[1] + 1e-16)
    h2a = o32[0] * recip + bias2_ref[:, 0:32]
    h2b = o32[1] * recip + bias2_ref[:, 32:64]
    bvec = batch_ref[0, :, :]
    gids = lax.broadcasted_iota(jnp.int32, (B, BLK_N), 0)
    oh = (gids == bvec).astype(_F32)
    sums_ref[:, 0:32] += jnp.dot(oh, h2a, preferred_element_type=_F32)
    sums_ref[:, 32:64] += jnp.dot(oh, h2b, preferred_element_type=_F32)
    cnt_ref[:, 0:1] += jnp.sum(oh, axis=1, keepdims=True)

    @pl.when(i == nblk - 1)
    def _fin():
        ge = sums_ref[...] / jnp.maximum(cnt_ref[:, 0:1], 1.0)
        bk = bk_ref[...]
        c = ge @ fw1_ref[0:64, :] + bk @ fw1_ref[64:65, :] + fb1_ref[...]
        c = jnp.maximum(c, 0.0)
        c = jnp.maximum(c @ fw2_ref[...] + fb2_ref[...], 0.0)
        out_ref[...] = bk + c @ fw3_ref[...] + fb3_ref[...]


def _full(shape):
    return pl.BlockSpec(shape, lambda i: tuple(0 for _ in shape))


def kernel(x, edge_index, batch, baseline_k, Wl1, bl1, Wr1, br1, att1, bias1,
           Wl2, bl2, Wr2, br2, att2, bias2, fw1, fb1, fw2, fb2, fw3, fb3):
    src = edge_index[0]
    dst = edge_index[1]

    kmask = np.zeros((H1, H1 * C1), np.float32)
    for h in range(H1):
        kmask[h, h * C1:(h + 1) * C1] = 1.0
    kmask_j = jnp.asarray(kmask)
    W4 = jnp.concatenate([Wl1, Wr1], axis=0)
    b4 = (bl1 + br1).reshape(1, H1 * C1)
    Asel = att1.reshape(H1 * C1, 1) * kmask_j.T
    rows = [bl1 * kmask[h] for h in range(H1)]
    for h in range(H1):
        for k in range(2):
            rows.append(Wl1[k] * kmask[h])
    for _ in range(4):
        rows.append(jnp.zeros((H1 * C1,), _F32))
    M16 = jnp.stack(rows, axis=0)
    rw = np.zeros((4, 8), np.float32)
    ru = np.zeros((4, 8), np.float32)
    for h in range(H1):
        for k in range(2):
            rw[h, 2 * h + k] = 1.0
            ru[k, 2 * h + k] = 1.0
    rw_j, ru_j = jnp.asarray(rw), jnp.asarray(ru)

    srcp = jnp.pad(src, (0, EP - E))
    dstp = jnp.pad(dst, (0, EP - E))
    src2d = srcp.reshape(EP // SCC, SCC)
    dst2d = dstp.reshape(EP // SCC, SCC)

    x8 = jnp.pad(x, ((0, 0), (0, 6)))
    usrc, udst = _gx(x8, src2d, dst2d)

    p = pl.pallas_call(
        _k1_edge1,
        grid=(EP // TBLK,),
        in_specs=[pl.BlockSpec((TBLK, 8), lambda i: (i, 0))] * 2 + [
            _full((4, H1 * C1)), _full((1, H1 * C1)),
            _full((H1 * C1, 4)), _full((4, 8)), _full((4, 8))],
        out_specs=pl.BlockSpec((TBLK, 16), lambda i: (i, 0)),
        out_shape=jax.ShapeDtypeStruct((EP, 16), _F32),
    )(usrc, udst, W4, b4, Asel, rw_j, ru_j)

    acc1 = _l1_scatter(p, dst2d)

    xl2a, xl2b, xr2a, xr2b = pl.pallas_call(
        _k2_node1,
        grid=(N // BLK_N,),
        in_specs=[pl.BlockSpec((NC, BLK_N, 16), lambda i: (0, i, 0)),
                  _full((16, H1 * C1)), _full((4, H1 * C1)), _full((1, H1 * C1)),
                  _full((H1 * C1, 32)), _full((H1 * C1, 32)),
                  _full((H1 * C1, 32)), _full((H1 * C1, 32)),
                  _full((1, C2)), _full((1, C2))],
        out_specs=[pl.BlockSpec((BLK_N, 32), lambda i: (i, 0))] * 4,
        out_shape=[jax.ShapeDtypeStruct((N, 32), _F32)] * 4,
    )(acc1, M16, kmask_j, bias1.reshape(1, -1),
      Wl2[:, 0:32], Wl2[:, 32:64], Wr2[:, 0:32], Wr2[:, 32:64],
      bl2.reshape(1, -1), br2.reshape(1, -1))

    logits = _l2_logits(xl2a, xl2b, xr2a, xr2b, att2.reshape(-1),
                        src2d, dst2d)

    o32 = _l2_scatter(xl2a, xl2b, logits, dst2d)
    od = _l2_scatterd(logits, dst2d)

    batch3d = batch.reshape(N // BLK_N, 1, BLK_N)
    od3 = od.reshape(NC, NPAD, 1)
    out = pl.pallas_call(
        _k4_finish,
        grid=(N // BLK_N,),
        in_specs=[pl.BlockSpec((NC, BLK_N, 32), lambda i: (0, i, 0)),
                  pl.BlockSpec((NC, BLK_N, 1), lambda i: (0, i, 0)),
                  pl.BlockSpec((1, 1, BLK_N), lambda i: (i, 0, 0)),
                  _full((1, C2)), _full((B, 1)),
                  _full((C2 + 1, 32)), _full((1, 32)),
                  _full((32, 16)), _full((1, 16)),
                  _full((16, 1)), _full((1, 1))],
        out_specs=pl.BlockSpec((B, 1), lambda i: (0, 0)),
        out_shape=jax.ShapeDtypeStruct((B, 1), _F32),
        scratch_shapes=[pltpu.VMEM((B, C2), _F32), pltpu.VMEM((B, 128), _F32)],
    )(o32, od3, batch3d, bias2.reshape(1, -1), baseline_k,
      fw1, fb1.reshape(1, -1), fw2, fb2.reshape(1, -1), fw3, fb3.reshape(1, -1))
    return out


# speedup vs baseline: 1.3256x; 1.0688x over previous
import functools
import jax
import jax.numpy as jnp
import numpy as np
from jax import lax
from jax.experimental import pallas as pl
from jax.experimental.pallas import tpu as pltpu
from jax.experimental.pallas import tpu_sc as plsc

N = 50000
E = 800000
B = 64
H1 = 4
C1 = 64
C2 = 64

EP = 819200
TBLK = 2048
BLK_N = 2000

NC = 2
NS = 16
SCC = 128
SROW = 8
NPAD = 51200
_SC_MESH = dict(core_axis_name="c", subcore_axis_name="s", num_cores=NC,
                num_subcores=NS)
_SC_PARAMS = pltpu.CompilerParams(use_tc_tiling_on_sc=False)

_F32 = jnp.float32


def _lane_sum(v):
    t = [v[j] for j in range(16)]
    while len(t) > 1:
        t = [t[a] + t[a + 1] for a in range(0, len(t), 2)]
    return t[0]


def _sc_gather_x(x_hbm, s2d_hbm, d2d_hbm, us_hbm, ud_hbm, ivs, ivd, us, ud,
                 sem):
    c = lax.axis_index("c")
    s = lax.axis_index("s")
    wid = c * NS + s
    e_pt = EP // (NC * NS)
    base = wid * e_pt

    def _outer(k, _):
        r0 = pl.multiple_of(base // SCC + k * SROW, 8)
        e0 = pl.multiple_of(base + k * SROW * SCC, 8)
        c1 = pltpu.async_copy(s2d_hbm.at[pl.ds(r0, SROW)], ivs, sem)
        c2 = pltpu.async_copy(d2d_hbm.at[pl.ds(r0, SROW)], ivd, sem)
        c1.wait()
        c2.wait()
        copies = []
        for j in range(SROW):
            copies.append(pltpu.async_copy(
                x_hbm.at[ivs.at[j]], us.at[pl.ds(j * SCC, SCC)], sem))
            copies.append(pltpu.async_copy(
                x_hbm.at[ivd.at[j]], ud.at[pl.ds(j * SCC, SCC)], sem))
        for cp in copies:
            cp.wait()
        pltpu.sync_copy(us, us_hbm.at[pl.ds(e0, SROW * SCC)])
        pltpu.sync_copy(ud, ud_hbm.at[pl.ds(e0, SROW * SCC)])
        return 0

    lax.fori_loop(0, e_pt // (SROW * SCC), _outer, 0)


_gx = functools.partial(
    pl.kernel,
    out_type=[jax.ShapeDtypeStruct((EP, 8), _F32) for _ in range(2)],
    mesh=plsc.VectorSubcoreMesh(**_SC_MESH),
    compiler_params=_SC_PARAMS,
    scratch_types=[
        pltpu.VMEM((SROW, SCC), jnp.int32),
        pltpu.VMEM((SROW, SCC), jnp.int32),
        pltpu.VMEM((SROW * SCC, 8), _F32),
        pltpu.VMEM((SROW * SCC, 8), _F32),
        pltpu.SemaphoreType.DMA,
    ],
)(_sc_gather_x)


def _sc_scatter16(p_hbm, dst_hbm, out_hbm, acc, iv, pv, rv):
    c = lax.axis_index("c")
    s = lax.axis_index("s")
    rows_pt = NPAD // NS
    e_pt = EP // (NC * NS)
    n_outer = e_pt // (SROW * SCC)

    def _zero_row(i, _):
        rv[i, :] = jnp.zeros((16,), _F32)
        return 0

    lax.fori_loop(0, rows_pt, _zero_row, 0)
    o0 = pl.multiple_of(s * rows_pt, 8)
    pltpu.sync_copy(rv, acc.at[pl.ds(o0, rows_pt)])
    plsc.subcore_barrier()

    base_e = (c * NS + s) * e_pt

    def _outer(k, _):
        r0 = pl.multiple_of(base_e // SCC + k * SROW, 8)
        e0 = pl.multiple_of(base_e + k * SROW * SCC, 8)
        pltpu.sync_copy(dst_hbm.at[pl.ds(r0, SROW)], iv)
        pltpu.sync_copy(p_hbm.at[pl.ds(e0, SROW * SCC)], pv)

        def _inner(j, _):
            pltpu.sync_copy(pv.at[pl.ds(j * SCC, SCC)], acc.at[iv.at[j]],
                            add=True)
            return 0

        lax.fori_loop(0, SROW, _inner, 0)
        return 0

    lax.fori_loop(0, n_outer, _outer, 0)
    plsc.subcore_barrier()
    pltpu.sync_copy(acc.at[pl.ds(o0, rows_pt)], rv)
    pltpu.sync_copy(rv, out_hbm.at[c, pl.ds(o0, rows_pt)])


_l1_scatter = functools.partial(
    pl.kernel,
    out_type=jax.ShapeDtypeStruct((NC, NPAD, 16), _F32),
    mesh=plsc.VectorSubcoreMesh(**_SC_MESH),
    compiler_params=_SC_PARAMS,
    scratch_types=[
        pltpu.VMEM_SHARED((NPAD, 16), _F32),
        pltpu.VMEM((SROW, SCC), jnp.int32),
        pltpu.VMEM((SROW * SCC, 16), _F32),
        pltpu.VMEM((NPAD // NS, 16), _F32),
    ],
)(_sc_scatter16)


def _sc_edge2(xl2a_hbm, xl2b_hbm, xr2a_hbm, xr2b_hbm, att_hbm, s2d_hbm,
              d2d_hbm, lo_hbm, attv, ivs, ivd, av0, av1, bv0, bv1, lv, sem):
    c = lax.axis_index("c")
    s = lax.axis_index("s")
    wid = c * NS + s
    e_pt = EP // (NC * NS)
    base = wid * e_pt
    pltpu.sync_copy(att_hbm, attv)

    def _outer(k, _):
        r0 = pl.multiple_of(base // SCC + k * SROW, 8)
        pltpu.sync_copy(s2d_hbm.at[pl.ds(r0, SROW)], ivs)
        pltpu.sync_copy(d2d_hbm.at[pl.ds(r0, SROW)], ivd)

        def _half(h, _):
            copies = []
            for j in range(4):
                row = h * 4 + j
                copies.append(pltpu.async_copy(
                    xl2a_hbm.at[ivs.at[row]], av0.at[pl.ds(j * SCC, SCC)], sem))
                copies.append(pltpu.async_copy(
                    xl2b_hbm.at[ivs.at[row]], av1.at[pl.ds(j * SCC, SCC)], sem))
                copies.append(pltpu.async_copy(
                    xr2a_hbm.at[ivd.at[row]], bv0.at[pl.ds(j * SCC, SCC)], sem))
                copies.append(pltpu.async_copy(
                    xr2b_hbm.at[ivd.at[row]], bv1.at[pl.ds(j * SCC, SCC)], sem))
            for cp in copies:
                cp.wait()

            def _grp(g, _):
                a0 = attv[pl.ds(0, 16)]
                a1 = attv[pl.ds(16, 16)]
                a2 = attv[pl.ds(32, 16)]
                a3 = attv[pl.ds(48, 16)]
                ioa = lax.iota(jnp.int32, 16)
                accv = jnp.zeros((16,), _F32)
                for i in range(16):
                    e = g * 16 + i
                    z0 = av0[e, pl.ds(0, 16)] + bv0[e, pl.ds(0, 16)]
                    z1 = av0[e, pl.ds(16, 16)] + bv0[e, pl.ds(16, 16)]
                    z2 = av1[e, pl.ds(0, 16)] + bv1[e, pl.ds(0, 16)]
                    z3 = av1[e, pl.ds(16, 16)] + bv1[e, pl.ds(16, 16)]
                    m = (jnp.where(z0 > 0, z0, 0.2 * z0) * a0
                         + jnp.where(z1 > 0, z1, 0.2 * z1) * a1
                         + jnp.where(z2 > 0, z2, 0.2 * z2) * a2
                         + jnp.where(z3 > 0, z3, 0.2 * z3) * a3)
                    logit = _lane_sum(m)
                    accv = jnp.where(ioa == i, jnp.full((16,), logit, _F32),
                                     accv)
                gidv = base + k * SROW * SCC + h * 512 + g * 16 + ioa
                lv[pl.ds(g * 16, 16)] = jnp.where(gidv < E, accv, -1e30)
                return 0

            lax.fori_loop(0, 32, _grp, 0)
            e0 = pl.multiple_of(base + k * SROW * SCC + h * 512, 8)
            pltpu.sync_copy(lv, lo_hbm.at[pl.ds(e0, 512)])
            return 0

        lax.fori_loop(0, 2, _half, 0)
        return 0

    lax.fori_loop(0, e_pt // (SROW * SCC), _outer, 0)


_l2_logits = functools.partial(
    pl.kernel,
    out_type=jax.ShapeDtypeStruct((EP,), _F32),
    mesh=plsc.VectorSubcoreMesh(**_SC_MESH),
    compiler_params=_SC_PARAMS,
    scratch_types=[
        pltpu.VMEM((C2,), _F32),
        pltpu.VMEM((SROW, SCC), jnp.int32),
        pltpu.VMEM((SROW, SCC), jnp.int32),
        pltpu.VMEM((512, 32), _F32),
        pltpu.VMEM((512, 32), _F32),
        pltpu.VMEM((512, 32), _F32),
        pltpu.VMEM((512, 32), _F32),
        pltpu.VMEM((512,), _F32),
        pltpu.SemaphoreType.DMA,
    ],
)(_sc_edge2)


ZCH = 160


def _sc_scatter2(xl2a_hbm, xl2b_hbm, lo_hbm, d2d_hbm, o32_hbm,
                 acc32, av, rv, lv, iv, zbuf, sem):
    c = lax.axis_index("c")
    s = lax.axis_index("s")
    rows_pt = NPAD // NS

    def _zrow(i, _):
        zbuf[i, pl.ds(0, 16)] = jnp.zeros((16,), _F32)
        zbuf[i, pl.ds(16, 16)] = jnp.zeros((16,), _F32)
        return 0

    lax.fori_loop(0, ZCH, _zrow, 0)
    o0 = pl.multiple_of(s * rows_pt, 8)
    for m in range(rows_pt // ZCH):
        pltpu.sync_copy(zbuf, acc32.at[pl.ds(o0 + m * ZCH, ZCH)])

    plsc.subcore_barrier()

    e_pt = EP // NS
    base = s * e_pt

    def _outer(k, _):
        r0 = pl.multiple_of(base // SCC + k * SROW, 8)
        e0 = pl.multiple_of(base + k * SROW * SCC, 8)
        c1 = pltpu.async_copy(d2d_hbm.at[pl.ds(r0, SROW)], iv, sem)
        c2 = pltpu.async_copy(lo_hbm.at[pl.ds(e0, SROW * SCC)], lv, sem)
        c1.wait()
        c2.wait()

        def _quarter(q, _):
            j0 = q * 2
            for jj in range(2):
                @pl.when(c == 0)
                def _():
                    pltpu.async_copy(xl2a_hbm.at[iv.at[j0 + jj]],
                                     av.at[pl.ds(jj * SCC, SCC)], sem)

                @pl.when(c == 1)
                def _():
                    pltpu.async_copy(xl2b_hbm.at[iv.at[j0 + jj]],
                                     av.at[pl.ds(jj * SCC, SCC)], sem)
            for jj in range(2):
                pltpu.make_async_copy(xl2a_hbm.at[iv.at[j0 + jj]],
                                      av.at[pl.ds(jj * SCC, SCC)], sem).wait()

            def _grp(g, _):
                wv = jnp.exp(lv[pl.ds(q * 2 * SCC + g * 16, 16)])
                for i in range(16):
                    e = g * 16 + i
                    wb = jnp.full((16,), wv[i], _F32)
                    rv[e, pl.ds(0, 16)] = av[e, pl.ds(0, 16)] * wb
                    rv[e, pl.ds(16, 16)] = av[e, pl.ds(16, 16)] * wb
                return 0

            lax.fori_loop(0, 2 * SCC // 16, _grp, 0)
            for jj in range(2):
                pltpu.sync_copy(rv.at[pl.ds(jj * SCC, SCC)],
                                acc32.at[iv.at[j0 + jj]], add=True)
            return 0

        lax.fori_loop(0, SROW // 2, _quarter, 0)
        return 0

    lax.fori_loop(0, e_pt // (SROW * SCC), _outer, 0)
    plsc.subcore_barrier()
    for m in range(rows_pt // ZCH):
        pltpu.sync_copy(acc32.at[pl.ds(o0 + m * ZCH, ZCH)], zbuf)
        pltpu.sync_copy(zbuf, o32_hbm.at[c, pl.ds(o0 + m * ZCH, ZCH)])


_l2_scatter = functools.partial(
    pl.kernel,
    out_type=jax.ShapeDtypeStruct((NC, NPAD, 32), _F32),
    mesh=plsc.VectorSubcoreMesh(**_SC_MESH),
    compiler_params=_SC_PARAMS,
    scratch_types=[
        pltpu.VMEM_SHARED((NPAD, 32), _F32),
        pltpu.VMEM((2 * SCC, 32), _F32),
        pltpu.VMEM((2 * SCC, 32), _F32),
        pltpu.VMEM((SROW * SCC,), _F32),
        pltpu.VMEM((SROW, SCC), jnp.int32),
        pltpu.VMEM((ZCH, 32), _F32),
        pltpu.SemaphoreType.DMA,
    ],
)(_sc_scatter2)


def _sc_scatterd(lo_hbm, d2d_hbm, od_hbm, accd, iv, lv, dv, rb):
    c = lax.axis_index("c")
    s = lax.axis_index("s")
    rows_pt = NPAD // NS

    def _zrow(i, _):
        rb[pl.ds(i * 16, 16)] = jnp.zeros((16,), _F32)
        return 0

    lax.fori_loop(0, rows_pt // 16, _zrow, 0)
    o0 = pl.multiple_of(s * rows_pt, 8)
    pltpu.sync_copy(rb, accd.at[pl.ds(o0, rows_pt)])
    plsc.subcore_barrier()

    e_pt = EP // (NC * NS)
    base = (c * NS + s) * e_pt

    def _outer(k, _):
        r0 = pl.multiple_of(base // SCC + k * SROW, 8)
        e0 = pl.multiple_of(base + k * SROW * SCC, 8)
        pltpu.sync_copy(d2d_hbm.at[pl.ds(r0, SROW)], iv)
        pltpu.sync_copy(lo_hbm.at[pl.ds(e0, SROW * SCC)], lv)

        def _grp(g, _):
            dv[pl.ds(g * 16, 16)] = jnp.exp(lv[pl.ds(g * 16, 16)])
            return 0

        lax.fori_loop(0, SROW * SCC // 16, _grp, 0)
        for j in range(SROW):
            pltpu.sync_copy(dv.at[pl.ds(j * SCC, SCC)], accd.at[iv.at[j]],
                            add=True)
        return 0

    lax.fori_loop(0, e_pt // (SROW * SCC), _outer, 0)
    plsc.subcore_barrier()
    pltpu.sync_copy(accd.at[pl.ds(o0, rows_pt)], rb)
    pltpu.sync_copy(rb, od_hbm.at[c, pl.ds(o0, rows_pt)])


_l2_scatterd = functools.partial(
    pl.kernel,
    out_type=jax.ShapeDtypeStruct((NC, NPAD), _F32),
    mesh=plsc.VectorSubcoreMesh(**_SC_MESH),
    compiler_params=_SC_PARAMS,
    scratch_types=[
        pltpu.VMEM_SHARED((NPAD,), _F32),
        pltpu.VMEM((SROW, SCC), jnp.int32),
        pltpu.VMEM((SROW * SCC,), _F32),
        pltpu.VMEM((SROW * SCC,), _F32),
        pltpu.VMEM((NPAD // NS,), _F32),
    ],
)(_sc_scatterd)


def _k1_edge1(us_ref, ud_ref, w4_ref, b4_ref, asel_ref, rw_ref, ru_ref, p_ref):
    i = pl.program_id(0)
    u = jnp.concatenate([us_ref[:, 0:2], ud_ref[:, 0:2]], axis=1)
    z = jnp.dot(u, w4_ref[...], preferred_element_type=_F32) + b4_ref[...]
    e = jnp.where(z > 0, z, 0.2 * z)
    logits = jnp.dot(e, asel_ref[...], preferred_element_type=_F32)
    rid = i * TBLK + lax.broadcasted_iota(jnp.int32, (TBLK, 1), 0)
    w = jnp.exp(logits) * (rid < E).astype(_F32)
    w_rep = jnp.dot(w, rw_ref[...], preferred_element_type=_F32)
    u_til = jnp.dot(u, ru_ref[...], preferred_element_type=_F32)
    p_ref[:, 0:4] = w
    p_ref[:, 4:12] = w_rep * u_til
    p_ref[:, 12:16] = jnp.zeros_like(p_ref[:, 12:16])


def _k2_node1(acc_ref, m16_ref, k4_ref, bias1_ref, wl2a_ref, wl2b_ref,
              wr2a_ref, wr2b_ref, bl2_ref, br2_ref,
              xl2a_ref, xl2b_ref, xr2a_ref, xr2b_ref):
    acc3 = acc_ref[...]
    acc = acc3[0] + acc3[1]
    denom = acc[:, 0:4]
    out1_pre = jnp.dot(acc, m16_ref[...], preferred_element_type=_F32)
    recip = 1.0 / (denom + 1e-16)
    bcast = jnp.dot(recip, k4_ref[...], preferred_element_type=_F32)
    h1 = jnp.maximum(out1_pre * bcast + bias1_ref[...], 0.0)
    xl2a_ref[...] = jnp.dot(h1, wl2a_ref[...], preferred_element_type=_F32) + bl2_ref[:, 0:32]
    xl2b_ref[...] = jnp.dot(h1, wl2b_ref[...], preferred_element_type=_F32) + bl2_ref[:, 32:64]
    xr2a_ref[...] = jnp.dot(h1, wr2a_ref[...], preferred_element_type=_F32) + br2_ref[:, 0:32]
    xr2b_ref[...] = jnp.dot(h1, wr2b_ref[...], preferred_element_type=_F32) + br2_ref[:, 32:64]


def _k4_finish(o32_ref, od_ref, batch_ref, bias2_ref, bk_ref, fw1_ref, fb1_ref,
               fw2_ref, fb2_ref, fw3_ref, fb3_ref, out_ref, sums_ref, cnt_ref):
    i = pl.program_id(0)
    nblk = pl.num_programs(0)

    @pl.when(i == 0)
    def _init():
        sums_ref[...] = jnp.zeros_like(sums_ref)
        cnt_ref[...] = jnp.zeros_like(cnt_ref)

    o32 = o32_ref[...]
    od3 = od_ref[...]
    recip = 1.0 / (od3[0] + od3[1] + 1e-16)
    h2a = o32[0] * recip + bias2_ref[:, 0:32]
    h2b = o32[1] * recip + bias2_ref[:, 32:64]
    bvec = batch_ref[0, :, :]
    gids = lax.broadcasted_iota(jnp.int32, (B, BLK_N), 0)
    oh = (gids == bvec).astype(_F32)
    sums_ref[:, 0:32] += jnp.dot(oh, h2a, preferred_element_type=_F32)
    sums_ref[:, 32:64] += jnp.dot(oh, h2b, preferred_element_type=_F32)
    cnt_ref[:, 0:1] += jnp.sum(oh, axis=1, keepdims=True)

    @pl.when(i == nblk - 1)
    def _fin():
        ge = sums_ref[...] / jnp.maximum(cnt_ref[:, 0:1], 1.0)
        bk = bk_ref[...]
        c = ge @ fw1_ref[0:64, :] + bk @ fw1_ref[64:65, :] + fb1_ref[...]
        c = jnp.maximum(c, 0.0)
        c = jnp.maximum(c @ fw2_ref[...] + fb2_ref[...], 0.0)
        out_ref[...] = bk + c @ fw3_ref[...] + fb3_ref[...]


def _full(shape):
    return pl.BlockSpec(shape, lambda i: tuple(0 for _ in shape))


def kernel(x, edge_index, batch, baseline_k, Wl1, bl1, Wr1, br1, att1, bias1,
           Wl2, bl2, Wr2, br2, att2, bias2, fw1, fb1, fw2, fb2, fw3, fb3):
    src = edge_index[0]
    dst = edge_index[1]

    kmask = np.zeros((H1, H1 * C1), np.float32)
    for h in range(H1):
        kmask[h, h * C1:(h + 1) * C1] = 1.0
    kmask_j = jnp.asarray(kmask)
    W4 = jnp.concatenate([Wl1, Wr1], axis=0)
    b4 = (bl1 + br1).reshape(1, H1 * C1)
    Asel = att1.reshape(H1 * C1, 1) * kmask_j.T
    rows = [bl1 * kmask[h] for h in range(H1)]
    for h in range(H1):
        for k in range(2):
            rows.append(Wl1[k] * kmask[h])
    for _ in range(4):
        rows.append(jnp.zeros((H1 * C1,), _F32))
    M16 = jnp.stack(rows, axis=0)
    rw = np.zeros((4, 8), np.float32)
    ru = np.zeros((4, 8), np.float32)
    for h in range(H1):
        for k in range(2):
            rw[h, 2 * h + k] = 1.0
            ru[k, 2 * h + k] = 1.0
    rw_j, ru_j = jnp.asarray(rw), jnp.asarray(ru)

    srcp = jnp.pad(src, (0, EP - E))
    dstp = jnp.pad(dst, (0, EP - E))
    src2d = srcp.reshape(EP // SCC, SCC)
    dst2d = dstp.reshape(EP // SCC, SCC)

    x8 = jnp.pad(x, ((0, 0), (0, 6)))
    usrc, udst = _gx(x8, src2d, dst2d)

    p = pl.pallas_call(
        _k1_edge1,
        grid=(EP // TBLK,),
        in_specs=[pl.BlockSpec((TBLK, 8), lambda i: (i, 0))] * 2 + [
            _full((4, H1 * C1)), _full((1, H1 * C1)),
            _full((H1 * C1, 4)), _full((4, 8)), _full((4, 8))],
        out_specs=pl.BlockSpec((TBLK, 16), lambda i: (i, 0)),
        out_shape=jax.ShapeDtypeStruct((EP, 16), _F32),
    )(usrc, udst, W4, b4, Asel, rw_j, ru_j)

    acc1 = _l1_scatter(p, dst2d)

    xl2a, xl2b, xr2a, xr2b = pl.pallas_call(
        _k2_node1,
        grid=(N // BLK_N,),
        in_specs=[pl.BlockSpec((NC, BLK_N, 16), lambda i: (0, i, 0)),
                  _full((16, H1 * C1)), _full((4, H1 * C1)), _full((1, H1 * C1)),
                  _full((H1 * C1, 32)), _full((H1 * C1, 32)),
                  _full((H1 * C1, 32)), _full((H1 * C1, 32)),
                  _full((1, C2)), _full((1, C2))],
        out_specs=[pl.BlockSpec((BLK_N, 32), lambda i: (i, 0))] * 4,
        out_shape=[jax.ShapeDtypeStruct((N, 32), _F32)] * 4,
    )(acc1, M16, kmask_j, bias1.reshape(1, -1),
      Wl2[:, 0:32], Wl2[:, 32:64], Wr2[:, 0:32], Wr2[:, 32:64],
      bl2.reshape(1, -1), br2.reshape(1, -1))

    logits = _l2_logits(xl2a, xl2b, xr2a, xr2b, att2.reshape(-1),
                        src2d, dst2d)

    o32 = _l2_scatter(xl2a, xl2b, logits, dst2d)
    od = _l2_scatterd(logits, dst2d)

    batch3d = batch.reshape(N // BLK_N, 1, BLK_N)
    od3 = od.reshape(NC, NPAD, 1)
    out = pl.pallas_call(
        _k4_finish,
        grid=(N // BLK_N,),
        in_specs=[pl.BlockSpec((NC, BLK_N, 32), lambda i: (0, i, 0)),
                  pl.BlockSpec((NC, BLK_N, 1), lambda i: (0, i, 0)),
                  pl.BlockSpec((1, 1, BLK_N), lambda i: (i, 0, 0)),
                  _full((1, C2)), _full((B, 1)),
                  _full((C2 + 1, 32)), _full((1, 32)),
                  _full((32, 16)), _full((1, 16)),
                  _full((16, 1)), _full((1, 1))],
        out_specs=pl.BlockSpec((B, 1), lambda i: (0, 0)),
        out_shape=jax.ShapeDtypeStruct((B, 1), _F32),
        scratch_shapes=[pltpu.VMEM((B, C2), _F32), pltpu.VMEM((B, 128), _F32)],
    )(o32, od3, batch3d, bias2.reshape(1, -1), baseline_k,
      fw1, fb1.reshape(1, -1), fw2, fb2.reshape(1, -1), fw3, fb3.reshape(1, -1))
    return out
